# Initial kernel scaffold; baseline (speedup 1.0000x reference)
import jax
import jax.numpy as jnp
from jax.experimental import pallas as pl


def kernel(x, edge_index, W_gc, b_gc, W_lin):
    raise NotImplementedError("write your pallas kernel here")


# speedup vs baseline: 4.1450x; 4.1450x over previous
import functools

import jax
import jax.numpy as jnp
from jax import lax
from jax.experimental import pallas as pl
from jax.experimental.pallas import tpu as pltpu
from jax.experimental.pallas import tpu_sc as plsc

N = 10000
D = 128
NCLS = 40
NITE = 4
SMOOTH = 0.5

NPAD = 10240
PAD_IDX = N
NW = 32
CH = 128
KCH = 80
QN = 5
QCH = KCH // QN
EPAD = NW * KCH * CH
RB = 1280
GRID = NPAD // RB
SROWS = NPAD // 16


def _wid():
    return lax.axis_index("s") * 2 + lax.axis_index("c")


def _spmm_body(sup_hbm, src_hbm, dst_hbm, zD_hbm, agg_hbm,
               src_v, dst_v, buf0, buf1, agg_sh, sem0, sem1):
    cid = lax.axis_index("c")
    sid = lax.axis_index("s")
    wid = _wid()

    pltpu.sync_copy(zD_hbm.at[pl.ds(sid * SROWS, SROWS)],
                    agg_sh.at[pl.ds(sid * SROWS, SROWS)])
    plsc.subcore_barrier()

    def quarter(q, _):
        pltpu.sync_copy(src_hbm.at[wid, pl.ds(q * QCH, QCH)], src_v)
        pltpu.sync_copy(dst_hbm.at[wid, pl.ds(q * QCH, QCH)], dst_v)
        pltpu.async_copy(sup_hbm.at[src_v.at[0]], buf0, sem0)

        def pair(j, _):
            pltpu.async_copy(sup_hbm.at[src_v.at[2 * j + 1]], buf1, sem1)
            pltpu.make_async_copy(sup_hbm.at[src_v.at[2 * j]], buf0,
                                  sem0).wait()
            pltpu.sync_copy(buf0, agg_sh.at[dst_v.at[2 * j]], add=True)
            pltpu.async_copy(sup_hbm.at[src_v.at[2 * j + 2]], buf0, sem0)
            pltpu.make_async_copy(sup_hbm.at[src_v.at[2 * j + 1]], buf1,
                                  sem1).wait()
            pltpu.sync_copy(buf1, agg_sh.at[dst_v.at[2 * j + 1]], add=True)
            return 0
        lax.fori_loop(0, QCH // 2 - 1, pair, 0)

        pltpu.async_copy(sup_hbm.at[src_v.at[QCH - 1]], buf1, sem1)
        pltpu.make_async_copy(sup_hbm.at[src_v.at[QCH - 2]], buf0,
                              sem0).wait()
        pltpu.sync_copy(buf0, agg_sh.at[dst_v.at[QCH - 2]], add=True)
        pltpu.make_async_copy(sup_hbm.at[src_v.at[QCH - 1]], buf1,
                              sem1).wait()
        pltpu.sync_copy(buf1, agg_sh.at[dst_v.at[QCH - 1]], add=True)
        return 0
    lax.fori_loop(0, QN, quarter, 0)
    plsc.subcore_barrier()

    pltpu.sync_copy(agg_sh.at[pl.ds(sid * SROWS, SROWS)],
                    agg_hbm.at[cid, pl.ds(sid * SROWS, SROWS)])


@functools.lru_cache(maxsize=1)
def _sc_kernels():
    mesh = plsc.VectorSubcoreMesh(core_axis_name="c", subcore_axis_name="s",
                                  num_cores=2, num_subcores=16)
    spmm = pl.kernel(
        _spmm_body,
        out_type=jax.ShapeDtypeStruct((2, NPAD, D), jnp.float32),
        mesh=mesh,
        scratch_types=[
            pltpu.VMEM((QCH, CH), jnp.int32),
            pltpu.VMEM((QCH, CH), jnp.int32),
            pltpu.VMEM((CH, D), jnp.float32),
            pltpu.VMEM((CH, D), jnp.float32),
            pltpu.VMEM_SHARED((NPAD, D), jnp.float32),
            pltpu.SemaphoreType.DMA,
            pltpu.SemaphoreType.DMA,
        ],
    )
    return spmm


def _col(dref):
    s = dref[0, :, :1] + dref[1, :, :1]
    return lax.rsqrt(jnp.maximum(s, 1.0))


def _kpre_body(x_ref, w_ref, dout_ref, sup_ref):
    f = _col(dout_ref)
    sup_ref[...] = jnp.dot(x_ref[...], w_ref[...],
                           preferred_element_type=jnp.float32) * f


def _kmid_body(h_ref, agg_ref, dout_ref, din_ref, w_ref, b_ref,
               h_out, sup_out):
    f = _col(dout_ref)
    g = _col(din_ref)
    agg = agg_ref[0] + agg_ref[1]
    aggn = g * agg + b_ref[...]
    hn = jnp.maximum(SMOOTH * h_ref[...] + (1.0 - SMOOTH) * aggn, 0.0)
    h_out[...] = hn
    sup_out[...] = jnp.dot(hn, w_ref[...],
                           preferred_element_type=jnp.float32) * f


def _kpost_body(h_ref, agg_ref, din_ref, b_ref, wl_ref, out_ref):
    g = _col(din_ref)
    agg = agg_ref[0] + agg_ref[1]
    aggn = g * agg + b_ref[...]
    hn = jnp.maximum(SMOOTH * h_ref[...] + (1.0 - SMOOTH) * aggn, 0.0)
    logits = jnp.dot(hn, wl_ref[...], preferred_element_type=jnp.float32)
    m = jnp.max(logits, axis=1, keepdims=True)
    lse = m + jnp.log(jnp.sum(jnp.exp(logits - m), axis=1, keepdims=True))
    out_ref[...] = logits - lse


def _row_spec():
    return pl.BlockSpec((RB, D), lambda i: (i, 0))


def _agg_spec():
    return pl.BlockSpec((2, RB, D), lambda i: (0, i, 0))


def kernel(x, edge_index, W_gc, b_gc, W_lin):
    src = edge_index[0]
    dst = edge_index[1]
    npad_e = EPAD - src.shape[0]
    pad = jnp.full((npad_e,), PAD_IDX, jnp.int32)
    src_p = jnp.concatenate([src, pad]).reshape(NW, KCH, CH)
    dst_p = jnp.concatenate([dst, pad]).reshape(NW, KCH, CH)
    x_p = jnp.pad(x, ((0, NPAD - N), (0, 0)))
    b2 = b_gc.reshape(1, D)
    zD = jnp.zeros((NPAD, D), jnp.float32)
    onesD = jnp.ones((NPAD, D), jnp.float32)

    spmm_kernel = _sc_kernels()
    dout = spmm_kernel(onesD, src_p, src_p, zD)
    din = spmm_kernel(onesD, dst_p, dst_p, zD)

    kpre = pl.pallas_call(
        _kpre_body,
        grid=(GRID,),
        in_specs=[
            _row_spec(),
            pl.BlockSpec((D, D), lambda i: (0, 0)),
            _agg_spec(),
        ],
        out_specs=_row_spec(),
        out_shape=jax.ShapeDtypeStruct((NPAD, D), jnp.float32),
    )
    sup = kpre(x_p, W_gc, dout)

    kmid = pl.pallas_call(
        _kmid_body,
        grid=(GRID,),
        in_specs=[
            _row_spec(),
            _agg_spec(),
            _agg_spec(),
            _agg_spec(),
            pl.BlockSpec((D, D), lambda i: (0, 0)),
            pl.BlockSpec((1, D), lambda i: (0, 0)),
        ],
        out_specs=[_row_spec(), _row_spec()],
        out_shape=[jax.ShapeDtypeStruct((NPAD, D), jnp.float32)] * 2,
    )

    h = x_p
    for _ in range(NITE - 1):
        agg = spmm_kernel(sup, src_p, dst_p, zD)
        h, sup = kmid(h, agg, dout, din, W_gc, b2)

    agg = spmm_kernel(sup, src_p, dst_p, zD)

    kpost = pl.pallas_call(
        _kpost_body,
        grid=(GRID,),
        in_specs=[
            _row_spec(),
            _agg_spec(),
            _agg_spec(),
            pl.BlockSpec((1, D), lambda i: (0, 0)),
            pl.BlockSpec((D, NCLS), lambda i: (0, 0)),
        ],
        out_specs=pl.BlockSpec((RB, NCLS), lambda i: (i, 0)),
        out_shape=jax.ShapeDtypeStruct((NPAD, NCLS), jnp.float32),
    )
    out = kpost(h, agg, din, b2, W_lin)
    return out[:N]


# speedup vs baseline: 4.3951x; 1.0603x over previous
import functools

import jax
import jax.numpy as jnp
from jax import lax
from jax.experimental import pallas as pl
from jax.experimental.pallas import tpu as pltpu
from jax.experimental.pallas import tpu_sc as plsc

N = 10000
D = 128
NCLS = 40
NITE = 4
SMOOTH = 0.5

NPAD = 10240
PAD_IDX = N
NW = 32
CH = 128
KCH = 80
QN = 5
QCH = KCH // QN
EPAD = NW * KCH * CH
RB = 1280
GRID = NPAD // RB
SROWS = NPAD // 16


def _wid():
    return lax.axis_index("s") * 2 + lax.axis_index("c")


def _spmm_body(sup_hbm, src_hbm, dst_hbm, zD_hbm, agg_hbm,
               src_v, dst_v, buf0, buf1, agg_sh, sem0, sem1, sem2, sem3):
    cid = lax.axis_index("c")
    sid = lax.axis_index("s")
    wid = _wid()

    pltpu.sync_copy(zD_hbm.at[pl.ds(sid * SROWS, SROWS)],
                    agg_sh.at[pl.ds(sid * SROWS, SROWS)])
    plsc.subcore_barrier()

    def _gather(row, buf, sem):
        pltpu.async_copy(sup_hbm.at[src_v.at[row]], buf, sem)

    def _gather_wait(row, buf, sem):
        pltpu.make_async_copy(sup_hbm.at[src_v.at[row]], buf, sem).wait()

    def _scatter(row, buf, sem):
        pltpu.async_copy(buf, agg_sh.at[dst_v.at[row]], sem, add=True)

    def _scatter_wait(row, buf, sem):
        pltpu.make_async_copy(buf, agg_sh.at[dst_v.at[row]], sem).wait()

    def quarter(q, _):
        pltpu.sync_copy(src_hbm.at[wid, pl.ds(q * QCH, QCH)], src_v)
        pltpu.sync_copy(dst_hbm.at[wid, pl.ds(q * QCH, QCH)], dst_v)
        _gather(0, buf0, sem0)
        _gather(1, buf1, sem1)

        def pair(j, _):
            a = 2 * j
            b = 2 * j + 1
            _gather_wait(a, buf0, sem0)
            _scatter(a, buf0, sem2)
            _gather_wait(b, buf1, sem1)
            _scatter(b, buf1, sem3)
            _scatter_wait(a, buf0, sem2)
            _gather(jnp.minimum(a + 2, QCH - 1), buf0, sem0)
            _scatter_wait(b, buf1, sem3)
            _gather(jnp.minimum(b + 2, QCH - 1), buf1, sem1)
            return 0
        lax.fori_loop(0, QCH // 2 - 1, pair, 0)

        a = QCH - 2
        b = QCH - 1
        _gather_wait(a, buf0, sem0)
        _scatter(a, buf0, sem2)
        _gather_wait(b, buf1, sem1)
        _scatter(b, buf1, sem3)
        _scatter_wait(a, buf0, sem2)
        _scatter_wait(b, buf1, sem3)
        return 0
    lax.fori_loop(0, QN, quarter, 0)
    plsc.subcore_barrier()

    pltpu.sync_copy(agg_sh.at[pl.ds(sid * SROWS, SROWS)],
                    agg_hbm.at[cid, pl.ds(sid * SROWS, SROWS)])


def _deg_body(src_hbm, dst_hbm, zD_hbm, olo_hbm, ohi_hbm, deg_hbm,
              src_v, dst_v, olo_v, ohi_v, acc_sh, sem2, sem3):
    cid = lax.axis_index("c")
    sid = lax.axis_index("s")
    wid = _wid()

    pltpu.sync_copy(zD_hbm.at[pl.ds(sid * SROWS, SROWS)],
                    acc_sh.at[pl.ds(sid * SROWS, SROWS)])
    pltpu.sync_copy(olo_hbm, olo_v)
    pltpu.sync_copy(ohi_hbm, ohi_v)
    plsc.subcore_barrier()

    def quarter(q, _):
        pltpu.sync_copy(src_hbm.at[wid, pl.ds(q * QCH, QCH)], src_v)
        pltpu.sync_copy(dst_hbm.at[wid, pl.ds(q * QCH, QCH)], dst_v)

        def fire(j, _):
            pltpu.async_copy(olo_v, acc_sh.at[src_v.at[j]], sem2, add=True)
            pltpu.async_copy(ohi_v, acc_sh.at[dst_v.at[j]], sem3, add=True)
            return 0
        lax.fori_loop(0, QCH, fire, 0)

        def drain(j, _):
            pltpu.make_async_copy(olo_v, acc_sh.at[src_v.at[j]], sem2).wait()
            pltpu.make_async_copy(ohi_v, acc_sh.at[dst_v.at[j]], sem3).wait()
            return 0
        lax.fori_loop(0, QCH, drain, 0)
        return 0
    lax.fori_loop(0, QN, quarter, 0)
    plsc.subcore_barrier()

    pltpu.sync_copy(acc_sh.at[pl.ds(sid * SROWS, SROWS)],
                    deg_hbm.at[cid, pl.ds(sid * SROWS, SROWS)])


@functools.lru_cache(maxsize=1)
def _sc_kernels():
    mesh = plsc.VectorSubcoreMesh(core_axis_name="c", subcore_axis_name="s",
                                  num_cores=2, num_subcores=16)
    spmm = pl.kernel(
        _spmm_body,
        out_type=jax.ShapeDtypeStruct((2, NPAD, D), jnp.float32),
        mesh=mesh,
        scratch_types=[
            pltpu.VMEM((QCH, CH), jnp.int32),
            pltpu.VMEM((QCH, CH), jnp.int32),
            pltpu.VMEM((CH, D), jnp.float32),
            pltpu.VMEM((CH, D), jnp.float32),
            pltpu.VMEM_SHARED((NPAD, D), jnp.float32),
            pltpu.SemaphoreType.DMA,
            pltpu.SemaphoreType.DMA,
            pltpu.SemaphoreType.DMA,
            pltpu.SemaphoreType.DMA,
        ],
    )
    deg = pl.kernel(
        _deg_body,
        out_type=jax.ShapeDtypeStruct((2, NPAD, D), jnp.float32),
        mesh=mesh,
        scratch_types=[
            pltpu.VMEM((QCH, CH), jnp.int32),
            pltpu.VMEM((QCH, CH), jnp.int32),
            pltpu.VMEM((CH, D), jnp.float32),
            pltpu.VMEM((CH, D), jnp.float32),
            pltpu.VMEM_SHARED((NPAD, D), jnp.float32),
            pltpu.SemaphoreType.DMA,
            pltpu.SemaphoreType.DMA,
        ],
    )
    return spmm, deg


def _col(dref, lane):
    s = dref[0, :, lane:lane + 1] + dref[1, :, lane:lane + 1]
    return lax.rsqrt(jnp.maximum(s, 1.0))


def _kpre_body(x_ref, w_ref, deg_ref, sup_ref):
    f = _col(deg_ref, 0)
    sup_ref[...] = jnp.dot(x_ref[...], w_ref[...],
                           preferred_element_type=jnp.float32) * f


def _kmid_body(h_ref, agg_ref, deg_ref, w_ref, b_ref,
               h_out, sup_out):
    f = _col(deg_ref, 0)
    g = _col(deg_ref, 64)
    agg = agg_ref[0] + agg_ref[1]
    aggn = g * agg + b_ref[...]
    hn = jnp.maximum(SMOOTH * h_ref[...] + (1.0 - SMOOTH) * aggn, 0.0)
    h_out[...] = hn
    sup_out[...] = jnp.dot(hn, w_ref[...],
                           preferred_element_type=jnp.float32) * f


def _kpost_body(h_ref, agg_ref, deg_ref, b_ref, wl_ref, out_ref):
    g = _col(deg_ref, 64)
    agg = agg_ref[0] + agg_ref[1]
    aggn = g * agg + b_ref[...]
    hn = jnp.maximum(SMOOTH * h_ref[...] + (1.0 - SMOOTH) * aggn, 0.0)
    logits = jnp.dot(hn, wl_ref[...], preferred_element_type=jnp.float32)
    m = jnp.max(logits, axis=1, keepdims=True)
    lse = m + jnp.log(jnp.sum(jnp.exp(logits - m), axis=1, keepdims=True))
    out_ref[...] = logits - lse


def _row_spec():
    return pl.BlockSpec((RB, D), lambda i: (i, 0))


def _agg_spec():
    return pl.BlockSpec((2, RB, D), lambda i: (0, i, 0))


def kernel(x, edge_index, W_gc, b_gc, W_lin):
    src = edge_index[0]
    dst = edge_index[1]
    npad_e = EPAD - src.shape[0]
    pad = jnp.full((npad_e,), PAD_IDX, jnp.int32)
    src_p = jnp.concatenate([src, pad]).reshape(NW, KCH, CH)
    dst_p = jnp.concatenate([dst, pad]).reshape(NW, KCH, CH)
    x_p = jnp.pad(x, ((0, NPAD - N), (0, 0)))
    b2 = b_gc.reshape(1, D)
    zD = jnp.zeros((NPAD, D), jnp.float32)
    lanes = jnp.arange(D)
    olo = jnp.broadcast_to((lanes < 64).astype(jnp.float32), (CH, D))
    ohi = jnp.broadcast_to((lanes >= 64).astype(jnp.float32), (CH, D))

    spmm_kernel, deg_kernel = _sc_kernels()
    degs = deg_kernel(src_p, dst_p, zD, olo, ohi)

    kpre = pl.pallas_call(
        _kpre_body,
        grid=(GRID,),
        in_specs=[
            _row_spec(),
            pl.BlockSpec((D, D), lambda i: (0, 0)),
            _agg_spec(),
        ],
        out_specs=_row_spec(),
        out_shape=jax.ShapeDtypeStruct((NPAD, D), jnp.float32),
    )
    sup = kpre(x_p, W_gc, degs)

    kmid = pl.pallas_call(
        _kmid_body,
        grid=(GRID,),
        in_specs=[
            _row_spec(),
            _agg_spec(),
            _agg_spec(),
            pl.BlockSpec((D, D), lambda i: (0, 0)),
            pl.BlockSpec((1, D), lambda i: (0, 0)),
        ],
        out_specs=[_row_spec(), _row_spec()],
        out_shape=[jax.ShapeDtypeStruct((NPAD, D), jnp.float32)] * 2,
    )

    h = x_p
    for _ in range(NITE - 1):
        agg = spmm_kernel(sup, src_p, dst_p, zD)
        h, sup = kmid(h, agg, degs, W_gc, b2)

    agg = spmm_kernel(sup, src_p, dst_p, zD)

    kpost = pl.pallas_call(
        _kpost_body,
        grid=(GRID,),
        in_specs=[
            _row_spec(),
            _agg_spec(),
            _agg_spec(),
            pl.BlockSpec((1, D), lambda i: (0, 0)),
            pl.BlockSpec((D, NCLS), lambda i: (0, 0)),
        ],
        out_specs=pl.BlockSpec((RB, NCLS), lambda i: (i, 0)),
        out_shape=jax.ShapeDtypeStruct((NPAD, NCLS), jnp.float32),
    )
    out = kpost(h, agg, degs, b2, W_lin)
    return out[:N]


# speedup vs baseline: 4.5036x; 1.0247x over previous
import functools

import jax
import jax.numpy as jnp
from jax import lax
from jax.experimental import pallas as pl
from jax.experimental.pallas import tpu as pltpu
from jax.experimental.pallas import tpu_sc as plsc

N = 10000
D = 128
NCLS = 40
NITE = 4
SMOOTH = 0.5

NPAD = 10240
PAD_IDX = N
NW = 32
CH = 128
KCH = 80
QN = 5
QCH = KCH // QN
EPAD = NW * KCH * CH
RB = 1280
GRID = NPAD // RB
SROWS = NPAD // 16


def _wid():
    return lax.axis_index("s") * 2 + lax.axis_index("c")


def _spmm_body(sup_hbm, src_hbm, dst_hbm, zD_hbm, agg_hbm,
               src_v, dst_v, buf0, buf1, agg_sh, sem0, sem1, sem2, sem3):
    cid = lax.axis_index("c")
    sid = lax.axis_index("s")
    wid = _wid()

    pltpu.sync_copy(zD_hbm.at[pl.ds(sid * SROWS, SROWS)],
                    agg_sh.at[pl.ds(sid * SROWS, SROWS)])
    plsc.subcore_barrier()

    def quarter(q, _):
        pltpu.sync_copy(src_hbm.at[wid, pl.ds(q * QCH, QCH)], src_v)
        pltpu.sync_copy(dst_hbm.at[wid, pl.ds(q * QCH, QCH)], dst_v)
        pltpu.async_copy(sup_hbm.at[src_v.at[0]], buf0, sem0)

        def pair(j, _):
            pltpu.async_copy(sup_hbm.at[src_v.at[2 * j + 1]], buf1, sem1)
            pltpu.make_async_copy(sup_hbm.at[src_v.at[2 * j]], buf0,
                                  sem0).wait()
            pltpu.sync_copy(buf0, agg_sh.at[dst_v.at[2 * j]], add=True)
            pltpu.async_copy(sup_hbm.at[src_v.at[2 * j + 2]], buf0, sem0)
            pltpu.make_async_copy(sup_hbm.at[src_v.at[2 * j + 1]], buf1,
                                  sem1).wait()
            pltpu.sync_copy(buf1, agg_sh.at[dst_v.at[2 * j + 1]], add=True)
            return 0
        lax.fori_loop(0, QCH // 2 - 1, pair, 0)

        pltpu.async_copy(sup_hbm.at[src_v.at[QCH - 1]], buf1, sem1)
        pltpu.make_async_copy(sup_hbm.at[src_v.at[QCH - 2]], buf0,
                              sem0).wait()
        pltpu.sync_copy(buf0, agg_sh.at[dst_v.at[QCH - 2]], add=True)
        pltpu.make_async_copy(sup_hbm.at[src_v.at[QCH - 1]], buf1,
                              sem1).wait()
        pltpu.sync_copy(buf1, agg_sh.at[dst_v.at[QCH - 1]], add=True)
        return 0
    lax.fori_loop(0, QN, quarter, 0)
    plsc.subcore_barrier()

    pltpu.sync_copy(agg_sh.at[pl.ds(sid * SROWS, SROWS)],
                    agg_hbm.at[cid, pl.ds(sid * SROWS, SROWS)])


def _deg_body(src_hbm, dst_hbm, zD_hbm, olo_hbm, ohi_hbm, deg_hbm,
              src_v, dst_v, olo_v, ohi_v, acc_sh, sem2, sem3):
    cid = lax.axis_index("c")
    sid = lax.axis_index("s")
    wid = _wid()

    pltpu.sync_copy(zD_hbm.at[pl.ds(sid * SROWS, SROWS)],
                    acc_sh.at[pl.ds(sid * SROWS, SROWS)])
    pltpu.sync_copy(olo_hbm, olo_v)
    pltpu.sync_copy(ohi_hbm, ohi_v)
    plsc.subcore_barrier()

    def quarter(q, _):
        pltpu.sync_copy(src_hbm.at[wid, pl.ds(q * QCH, QCH)], src_v)
        pltpu.sync_copy(dst_hbm.at[wid, pl.ds(q * QCH, QCH)], dst_v)

        def fire(j, _):
            pltpu.async_copy(olo_v, acc_sh.at[src_v.at[j]], sem2, add=True)
            pltpu.async_copy(ohi_v, acc_sh.at[dst_v.at[j]], sem3, add=True)
            return 0
        lax.fori_loop(0, QCH, fire, 0)

        def drain(j, _):
            pltpu.make_async_copy(olo_v, acc_sh.at[src_v.at[j]], sem2).wait()
            pltpu.make_async_copy(ohi_v, acc_sh.at[dst_v.at[j]], sem3).wait()
            return 0
        lax.fori_loop(0, QCH, drain, 0)
        return 0
    lax.fori_loop(0, QN, quarter, 0)
    plsc.subcore_barrier()

    pltpu.sync_copy(acc_sh.at[pl.ds(sid * SROWS, SROWS)],
                    deg_hbm.at[cid, pl.ds(sid * SROWS, SROWS)])


@functools.lru_cache(maxsize=1)
def _sc_kernels():
    mesh = plsc.VectorSubcoreMesh(core_axis_name="c", subcore_axis_name="s",
                                  num_cores=2, num_subcores=16)
    spmm = pl.kernel(
        _spmm_body,
        out_type=jax.ShapeDtypeStruct((2, NPAD, D), jnp.float32),
        mesh=mesh,
        scratch_types=[
            pltpu.VMEM((QCH, CH), jnp.int32),
            pltpu.VMEM((QCH, CH), jnp.int32),
            pltpu.VMEM((CH, D), jnp.float32),
            pltpu.VMEM((CH, D), jnp.float32),
            pltpu.VMEM_SHARED((NPAD, D), jnp.float32),
            pltpu.SemaphoreType.DMA,
            pltpu.SemaphoreType.DMA,
            pltpu.SemaphoreType.DMA,
            pltpu.SemaphoreType.DMA,
        ],
    )
    deg = pl.kernel(
        _deg_body,
        out_type=jax.ShapeDtypeStruct((2, NPAD, D), jnp.float32),
        mesh=mesh,
        scratch_types=[
            pltpu.VMEM((QCH, CH), jnp.int32),
            pltpu.VMEM((QCH, CH), jnp.int32),
            pltpu.VMEM((CH, D), jnp.float32),
            pltpu.VMEM((CH, D), jnp.float32),
            pltpu.VMEM_SHARED((NPAD, D), jnp.float32),
            pltpu.SemaphoreType.DMA,
            pltpu.SemaphoreType.DMA,
        ],
    )
    return spmm, deg


def _col(dref, lane):
    s = dref[0, :, lane:lane + 1] + dref[1, :, lane:lane + 1]
    return lax.rsqrt(jnp.maximum(s, 1.0))


def _kpre_body(x_ref, w_ref, deg_ref, sup_ref):
    f = _col(deg_ref, 0)
    sup_ref[...] = jnp.dot(x_ref[...], w_ref[...],
                           preferred_element_type=jnp.float32) * f


def _kmid_body(h_ref, agg_ref, deg_ref, w_ref, b_ref,
               h_out, sup_out):
    f = _col(deg_ref, 0)
    g = _col(deg_ref, 64)
    agg = agg_ref[0] + agg_ref[1]
    aggn = g * agg + b_ref[...]
    hn = jnp.maximum(SMOOTH * h_ref[...] + (1.0 - SMOOTH) * aggn, 0.0)
    h_out[...] = hn
    sup_out[...] = jnp.dot(hn, w_ref[...],
                           preferred_element_type=jnp.float32) * f


def _kpost_body(h_ref, agg_ref, deg_ref, b_ref, wl_ref, out_ref):
    g = _col(deg_ref, 64)
    agg = agg_ref[0] + agg_ref[1]
    aggn = g * agg + b_ref[...]
    hn = jnp.maximum(SMOOTH * h_ref[...] + (1.0 - SMOOTH) * aggn, 0.0)
    logits = jnp.dot(hn, wl_ref[...], preferred_element_type=jnp.float32)
    m = jnp.max(logits, axis=1, keepdims=True)
    lse = m + jnp.log(jnp.sum(jnp.exp(logits - m), axis=1, keepdims=True))
    out_ref[...] = logits - lse


def _row_spec():
    return pl.BlockSpec((RB, D), lambda i: (i, 0))


def _agg_spec():
    return pl.BlockSpec((2, RB, D), lambda i: (0, i, 0))


def kernel(x, edge_index, W_gc, b_gc, W_lin):
    src = edge_index[0]
    dst = edge_index[1]
    npad_e = EPAD - src.shape[0]
    pad = jnp.full((npad_e,), PAD_IDX, jnp.int32)
    src_p = jnp.concatenate([src, pad]).reshape(NW, KCH, CH)
    dst_p = jnp.concatenate([dst, pad]).reshape(NW, KCH, CH)
    x_p = jnp.pad(x, ((0, NPAD - N), (0, 0)))
    b2 = b_gc.reshape(1, D)
    zD = jnp.zeros((NPAD, D), jnp.float32)
    lanes = jnp.arange(D)
    olo = jnp.broadcast_to((lanes < 64).astype(jnp.float32), (CH, D))
    ohi = jnp.broadcast_to((lanes >= 64).astype(jnp.float32), (CH, D))

    spmm_kernel, deg_kernel = _sc_kernels()
    degs = deg_kernel(src_p, dst_p, zD, olo, ohi)

    kpre = pl.pallas_call(
        _kpre_body,
        grid=(GRID,),
        in_specs=[
            _row_spec(),
            pl.BlockSpec((D, D), lambda i: (0, 0)),
            _agg_spec(),
        ],
        out_specs=_row_spec(),
        out_shape=jax.ShapeDtypeStruct((NPAD, D), jnp.float32),
    )
    sup = kpre(x_p, W_gc, degs)

    kmid = pl.pallas_call(
        _kmid_body,
        grid=(GRID,),
        in_specs=[
            _row_spec(),
            _agg_spec(),
            _agg_spec(),
            pl.BlockSpec((D, D), lambda i: (0, 0)),
            pl.BlockSpec((1, D), lambda i: (0, 0)),
        ],
        out_specs=[_row_spec(), _row_spec()],
        out_shape=[jax.ShapeDtypeStruct((NPAD, D), jnp.float32)] * 2,
    )

    h = x_p
    for _ in range(NITE - 1):
        agg = spmm_kernel(sup, src_p, dst_p, zD)
        h, sup = kmid(h, agg, degs, W_gc, b2)

    agg = spmm_kernel(sup, src_p, dst_p, zD)

    kpost = pl.pallas_call(
        _kpost_body,
        grid=(GRID,),
        in_specs=[
            _row_spec(),
            _agg_spec(),
            _agg_spec(),
            pl.BlockSpec((1, D), lambda i: (0, 0)),
            pl.BlockSpec((D, NCLS), lambda i: (0, 0)),
        ],
        out_specs=pl.BlockSpec((RB, NCLS), lambda i: (i, 0)),
        out_shape=jax.ShapeDtypeStruct((NPAD, NCLS), jnp.float32),
    )
    out = kpost(h, agg, degs, b2, W_lin)
    return out[:N]
